# baseline (speedup 1.0000x reference)
import jax
import jax.numpy as jnp
from jax.experimental import pallas as pl

_BT = 1024


def _router_block(x_ref, wt_ref, b_ref, o_ref):
    xb = x_ref[...].astype(jnp.bfloat16)
    acc = jax.lax.dot_general(
        xb, wt_ref[...], (((1,), (0,)), ((), ())),
        preferred_element_type=jnp.float32)
    o_ref[...] = acc + b_ref[...]


def kernel(x, W, b):
    tokens, d = x.shape
    e = W.shape[0]
    wt = W.T.astype(jnp.bfloat16)
    b2 = b.reshape(1, e)
    return pl.pallas_call(
        _router_block,
        grid=(tokens // _BT,),
        in_specs=[
            pl.BlockSpec((_BT, d), lambda i: (i, 0)),
            pl.BlockSpec((d, e), lambda i: (0, 0)),
            pl.BlockSpec((1, e), lambda i: (0, 0)),
        ],
        out_specs=pl.BlockSpec((_BT, e), lambda i: (i, 0)),
        out_shape=jax.ShapeDtypeStruct((tokens, e), jnp.float32),
    )(x, wt, b2)


# speedup vs baseline: 1.0205x; 1.0205x over previous
import jax
import jax.numpy as jnp
from jax.experimental import pallas as pl

_BT = 1024


def _router_block(x_ref, w_ref, b_ref, o_ref):
    xb = x_ref[...].astype(jnp.bfloat16)
    wb = w_ref[...].astype(jnp.bfloat16)
    acc = jax.lax.dot_general(
        xb, wb, (((1,), (1,)), ((), ())),
        preferred_element_type=jnp.float32)
    o_ref[...] = acc + b_ref[...]


def kernel(x, W, b):
    tokens, d = x.shape
    e = W.shape[0]
    b2 = b.reshape(1, e)
    return pl.pallas_call(
        _router_block,
        grid=(tokens // _BT,),
        in_specs=[
            pl.BlockSpec((_BT, d), lambda i: (i, 0)),
            pl.BlockSpec((e, d), lambda i: (0, 0)),
            pl.BlockSpec((1, e), lambda i: (0, 0)),
        ],
        out_specs=pl.BlockSpec((_BT, e), lambda i: (i, 0)),
        out_shape=jax.ShapeDtypeStruct((tokens, e), jnp.float32),
    )(x, W, b2)


# speedup vs baseline: 1.1187x; 1.0962x over previous
import jax
import jax.numpy as jnp
from jax.experimental import pallas as pl

_BT = 1024


def _router_block(x_ref, w_ref, b_ref, o_ref):
    xb = x_ref[...].astype(jnp.bfloat16)
    wb = w_ref[...].astype(jnp.bfloat16)
    acc = jax.lax.dot_general(
        wb, xb, (((1,), (1,)), ((), ())),
        preferred_element_type=jnp.float32)
    o_ref[...] = acc + b_ref[...]


def kernel(x, W, b):
    tokens, d = x.shape
    e = W.shape[0]
    b2 = b.reshape(e, 1)
    logits_t = pl.pallas_call(
        _router_block,
        grid=(tokens // _BT,),
        in_specs=[
            pl.BlockSpec((_BT, d), lambda i: (i, 0)),
            pl.BlockSpec((e, d), lambda i: (0, 0)),
            pl.BlockSpec((e, 1), lambda i: (0, 0)),
        ],
        out_specs=pl.BlockSpec((e, _BT), lambda i: (0, i)),
        out_shape=jax.ShapeDtypeStruct((e, tokens), jnp.float32),
    )(x, W, b2)
    return logits_t.T
